# baseline (speedup 1.0000x reference)
import functools

import jax
import jax.numpy as jnp
from jax import lax
from jax.experimental import pallas as pl
from jax.experimental.pallas import tpu as pltpu
from jax.experimental.pallas import tpu_sc as plsc


def _make_depad(V, E, NC, NS, LANES):
    NW = NC * NS
    R = 256
    n_full = V // R
    tail = V - n_full * R
    rounds = n_full // NW
    leftover = n_full - rounds * NW
    assert rounds % 2 == 0 and tail % 8 == 0 and E % LANES == 0
    n_pairs = rounds // 2
    n_grp = E // LANES

    mesh = plsc.VectorSubcoreMesh(core_axis_name="c", subcore_axis_name="s")

    @functools.partial(
        pl.kernel,
        out_type=jax.ShapeDtypeStruct((V * E,), jnp.float32),
        mesh=mesh,
        compiler_params=pltpu.CompilerParams(
            use_tc_tiling_on_sc=True, needs_layout_passes=False),
        scratch_types=[
            pltpu.VMEM((R, E), jnp.float32),
            pltpu.VMEM((R, E), jnp.float32),
            pltpu.VMEM((R * E,), jnp.float32),
            pltpu.VMEM((R * E,), jnp.float32),
            pltpu.SemaphoreType.DMA,
            pltpu.SemaphoreType.DMA,
            pltpu.SemaphoreType.DMA,
            pltpu.SemaphoreType.DMA,
        ],
    )
    def depad(tab_hbm, out_hbm, in0, in1, ou0, ou1, si0, si1, so0, so1):
        wid = lax.axis_index("s") * NC + lax.axis_index("c")
        ins = (in0, in1)
        ous = (ou0, ou1)
        sis = (si0, si1)
        sos = (so0, so1)

        def in_desc(k, j):
            r0 = (k * NW + wid) * R
            return pltpu.make_async_copy(
                tab_hbm.at[pl.ds(r0, R)], ins[j], sis[j]
            )

        def out_desc(k, j):
            r0 = (k * NW + wid) * R
            return pltpu.make_async_copy(
                ous[j], out_hbm.at[pl.ds(r0 * E, R * E)], sos[j]
            )

        def flatten(j, rows):
            src = ins[j]
            dst = ous[j]

            @plsc.parallel_loop(0, rows, unroll=8)
            def _(r):
                for c in range(n_grp):
                    dst[pl.ds(r * E + c * LANES, LANES)] = (
                        src[r, pl.ds(c * LANES, LANES)])

        in_desc(0, 0).start()

        def body(p, carry):
            k0 = 2 * p
            in_desc(k0 + 1, 1).start()
            in_desc(k0, 0).wait()

            @pl.when(p > 0)
            def _():
                out_desc(k0, 0).wait()

            flatten(0, R)
            out_desc(k0, 0).start()

            @pl.when(p < n_pairs - 1)
            def _():
                in_desc(k0 + 2, 0).start()

            in_desc(k0 + 1, 1).wait()

            @pl.when(p > 0)
            def _():
                out_desc(k0 + 1, 1).wait()

            flatten(1, R)
            out_desc(k0 + 1, 1).start()
            return carry

        lax.fori_loop(0, n_pairs, body, 0)
        out_desc(0, 0).wait()
        out_desc(1, 1).wait()

        @pl.when(wid < leftover)
        def _():
            in_desc(rounds, 0).start()
            in_desc(rounds, 0).wait()
            flatten(0, R)
            out_desc(rounds, 0).start()
            out_desc(rounds, 0).wait()

        if tail:
            @pl.when(wid == leftover)
            def _():
                r0 = n_full * R
                pltpu.make_async_copy(
                    tab_hbm.at[pl.ds(r0, tail)],
                    in0.at[pl.ds(0, tail)], si0,
                ).start()
                pltpu.make_async_copy(
                    tab_hbm.at[pl.ds(r0, tail)],
                    in0.at[pl.ds(0, tail)], si0,
                ).wait()
                flatten(0, tail)
                pltpu.make_async_copy(
                    ou0.at[pl.ds(0, tail * E)],
                    out_hbm.at[pl.ds(r0 * E, tail * E)], so0,
                ).start()
                pltpu.make_async_copy(
                    ou0.at[pl.ds(0, tail * E)],
                    out_hbm.at[pl.ds(r0 * E, tail * E)], so0,
                ).wait()

    return depad


def _make_pool(B, L, E, NC, NS, LANES):
    NW = NC * NS
    assert B % NW == 0 and E % LANES == 0
    b_per_w = B // NW
    assert b_per_w % 2 == 0
    n_pairs = b_per_w // 2
    n_acc = E // LANES
    chunks = []
    off = 0
    while off < L:
        n = min(128, L - off)
        chunks.append((off, n))
        off += n

    mesh = plsc.VectorSubcoreMesh(core_axis_name="c", subcore_axis_name="s")

    @functools.partial(
        pl.kernel,
        out_type=jax.ShapeDtypeStruct((B, E), jnp.float32),
        mesh=mesh,
        compiler_params=pltpu.CompilerParams(use_tc_tiling_on_sc=False),
        scratch_types=[
            pltpu.VMEM((b_per_w, L), jnp.int32),
            pltpu.VMEM((L, E), jnp.float32),
            pltpu.VMEM((L, E), jnp.float32),
            pltpu.VMEM((b_per_w, E), jnp.float32),
            pltpu.SemaphoreType.DMA,
            pltpu.SemaphoreType.DMA,
        ],
    )
    def pool(ids_hbm, emb_hbm, out_hbm, idx_v, buf0, buf1, out_v, s0, s1):
        wid = lax.axis_index("s") * NC + lax.axis_index("c")
        base = wid * b_per_w
        bufs = (buf0, buf1)
        sems = (s0, s1)

        pltpu.sync_copy(ids_hbm.at[pl.ds(base, b_per_w)], idx_v)

        def descs(b, k):
            return [
                pltpu.make_async_copy(
                    emb_hbm.at[idx_v.at[b, pl.ds(off, n)]],
                    bufs[k].at[pl.ds(off, n)],
                    sems[k],
                )
                for off, n in chunks
            ]

        def issue(b, k):
            for cp in descs(b, k):
                cp.start()

        def drain(b, k):
            for cp in descs(b, k):
                cp.wait()

        def accumulate(b, k):
            buf = bufs[k]

            def acc_body(j, accs):
                return tuple(
                    accs[c] + buf[j, pl.ds(c * LANES, LANES)]
                    for c in range(n_acc)
                )

            accs = tuple(
                jnp.zeros((LANES,), jnp.float32) for _ in range(n_acc)
            )
            accs = lax.fori_loop(0, L, acc_body, accs, unroll=8)
            scale = jnp.float32(1.0 / L)
            for c in range(n_acc):
                out_v[b, pl.ds(c * LANES, LANES)] = accs[c] * scale

        issue(0, 0)

        def body(g, carry):
            b0 = 2 * g
            issue(b0 + 1, 1)
            drain(b0, 0)
            accumulate(b0, 0)

            @pl.when(g < n_pairs - 1)
            def _():
                issue(b0 + 2, 0)

            drain(b0 + 1, 1)
            accumulate(b0 + 1, 1)
            return carry

        lax.fori_loop(0, n_pairs, body, 0)
        pltpu.sync_copy(out_v, out_hbm.at[pl.ds(base, b_per_w)])

    return pool


def _mlp(pooled, W1, b1, W2, b2):
    B, E = pooled.shape
    H = W1.shape[0]
    O = W2.shape[0]
    OP = 128
    W2p = jnp.zeros((OP, H), W2.dtype).at[:O].set(W2)
    b2p = jnp.zeros((1, OP), b2.dtype).at[0, :O].set(b2)
    b1r = b1.reshape(1, H)
    BLK = 1024

    def body(x_ref, w1_ref, b1_ref, w2_ref, b2_ref, o_ref):
        x = x_ref[...]
        h = lax.dot_general(
            x, w1_ref[...], (((1,), (1,)), ((), ())),
            preferred_element_type=jnp.float32,
        ) + b1_ref[...]
        h = jnp.maximum(h, 0.0)
        o_ref[...] = lax.dot_general(
            h, w2_ref[...], (((1,), (1,)), ((), ())),
            preferred_element_type=jnp.float32,
        ) + b2_ref[...]

    out = pl.pallas_call(
        body,
        out_shape=jax.ShapeDtypeStruct((B, OP), jnp.float32),
        grid=(B // BLK,),
        in_specs=[
            pl.BlockSpec((BLK, E), lambda i: (i, 0)),
            pl.BlockSpec((H, E), lambda i: (0, 0)),
            pl.BlockSpec((1, H), lambda i: (0, 0)),
            pl.BlockSpec((OP, H), lambda i: (0, 0)),
            pl.BlockSpec((1, OP), lambda i: (0, 0)),
        ],
        out_specs=pl.BlockSpec((BLK, OP), lambda i: (i, 0)),
    )(pooled, W1, b1r, W2p, b2p)
    return out[:, :O]


def kernel(input_ids, emb, W1, b1, W2, b2):
    B, L = input_ids.shape
    V, E = emb.shape
    info = plsc.get_sparse_core_info()
    NC, NS, LANES = info.num_cores, info.num_subcores, info.num_lanes
    flat = _make_depad(V, E, NC, NS, LANES)(emb)
    emb_dense = flat.reshape(V, E)
    pool = _make_pool(B, L, E, NC, NS, LANES)
    pooled = pool(input_ids.astype(jnp.int32), emb_dense)
    return _mlp(pooled, W1, b1, W2, b2)


# speedup vs baseline: 1.1754x; 1.1754x over previous
import functools

import jax
import jax.numpy as jnp
from jax import lax
from jax.experimental import pallas as pl
from jax.experimental.pallas import tpu as pltpu
from jax.experimental.pallas import tpu_sc as plsc


def _make_pool(B, L, E, NC, NS, LANES):
    NW = NC * NS
    assert B % NW == 0 and E % LANES == 0
    b_per_w = B // NW
    NBUF = 4
    assert b_per_w % NBUF == 0
    n_quads = b_per_w // NBUF
    n_acc = E // LANES
    chunks = []
    off = 0
    while off < L:
        n = min(128, L - off)
        chunks.append((off, n))
        off += n

    mesh = plsc.VectorSubcoreMesh(core_axis_name="c", subcore_axis_name="s")

    @functools.partial(
        pl.kernel,
        out_type=jax.ShapeDtypeStruct((B, E), jnp.float32),
        mesh=mesh,
        compiler_params=pltpu.CompilerParams(use_tc_tiling_on_sc=False),
        scratch_types=[
            pltpu.VMEM((b_per_w, L), jnp.int32),
            pltpu.VMEM((L, E), jnp.float32),
            pltpu.VMEM((L, E), jnp.float32),
            pltpu.VMEM((L, E), jnp.float32),
            pltpu.VMEM((L, E), jnp.float32),
            pltpu.VMEM((b_per_w, E), jnp.float32),
            pltpu.SemaphoreType.DMA,
            pltpu.SemaphoreType.DMA,
            pltpu.SemaphoreType.DMA,
            pltpu.SemaphoreType.DMA,
        ],
    )
    def pool(ids_hbm, emb_hbm, out_hbm, idx_v,
             buf0, buf1, buf2, buf3, out_v, s0, s1, s2, s3):
        wid = lax.axis_index("s") * NC + lax.axis_index("c")
        base = wid * b_per_w
        bufs = (buf0, buf1, buf2, buf3)
        sems = (s0, s1, s2, s3)

        pltpu.sync_copy(ids_hbm.at[pl.ds(base, b_per_w)], idx_v)

        def descs(b, k):
            return [
                pltpu.make_async_copy(
                    emb_hbm.at[idx_v.at[b, pl.ds(off, n)]],
                    bufs[k].at[pl.ds(off, n)],
                    sems[k],
                )
                for off, n in chunks
            ]

        def issue(b, k):
            for cp in descs(b, k):
                cp.start()

        def drain(b, k):
            for cp in descs(b, k):
                cp.wait()

        def accumulate(b, k):
            buf = bufs[k]

            def acc_body(j, accs):
                return tuple(
                    accs[c] + buf[j, pl.ds(c * LANES, LANES)]
                    for c in range(n_acc)
                )

            accs = tuple(
                jnp.zeros((LANES,), jnp.float32) for _ in range(n_acc)
            )
            accs = lax.fori_loop(0, L, acc_body, accs, unroll=8)
            scale = jnp.float32(1.0 / L)
            for c in range(n_acc):
                out_v[b, pl.ds(c * LANES, LANES)] = accs[c] * scale

        for k in range(NBUF - 1):
            issue(k, k)

        def body(g, carry):
            b0 = NBUF * g
            for k in range(NBUF):
                b = b0 + k
                drain(b, k)

                @pl.when(b + NBUF - 1 < b_per_w)
                def _():
                    issue(b + NBUF - 1, (k + NBUF - 1) % NBUF)

                accumulate(b, k)
            return carry

        lax.fori_loop(0, n_quads, body, 0)
        pltpu.sync_copy(out_v, out_hbm.at[pl.ds(base, b_per_w)])

    return pool


def _mlp(pooled, W1, b1, W2, b2):
    B, E = pooled.shape
    H = W1.shape[0]
    O = W2.shape[0]
    OP = 128
    W2p = jnp.zeros((OP, H), W2.dtype).at[:O].set(W2)
    b2p = jnp.zeros((1, OP), b2.dtype).at[0, :O].set(b2)
    b1r = b1.reshape(1, H)
    BLK = 1024

    def body(x_ref, w1_ref, b1_ref, w2_ref, b2_ref, o_ref):
        x = x_ref[...]
        h = lax.dot_general(
            x, w1_ref[...], (((1,), (1,)), ((), ())),
            preferred_element_type=jnp.float32,
        ) + b1_ref[...]
        h = jnp.maximum(h, 0.0)
        o_ref[...] = lax.dot_general(
            h, w2_ref[...], (((1,), (1,)), ((), ())),
            preferred_element_type=jnp.float32,
        ) + b2_ref[...]

    out = pl.pallas_call(
        body,
        out_shape=jax.ShapeDtypeStruct((B, OP), jnp.float32),
        grid=(B // BLK,),
        in_specs=[
            pl.BlockSpec((BLK, E), lambda i: (i, 0)),
            pl.BlockSpec((H, E), lambda i: (0, 0)),
            pl.BlockSpec((1, H), lambda i: (0, 0)),
            pl.BlockSpec((OP, H), lambda i: (0, 0)),
            pl.BlockSpec((1, OP), lambda i: (0, 0)),
        ],
        out_specs=pl.BlockSpec((BLK, OP), lambda i: (i, 0)),
    )(pooled, W1, b1r, W2p, b2p)
    return out[:, :O]


def kernel(input_ids, emb, W1, b1, W2, b2):
    B, L = input_ids.shape
    V, E = emb.shape
    info = plsc.get_sparse_core_info()
    NC, NS, LANES = info.num_cores, info.num_subcores, info.num_lanes
    embp = jnp.pad(emb, ((0, 0), (0, 128 - E)))
    emb2 = embp.reshape(2 * V, E)
    pool = _make_pool(B, L, E, NC, NS, LANES)
    pooled = pool(input_ids.astype(jnp.int32) * 2, emb2)
    return _mlp(pooled, W1, b1, W2, b2)


# speedup vs baseline: 1.1771x; 1.0014x over previous
import functools

import jax
import jax.numpy as jnp
from jax import lax
from jax.experimental import pallas as pl
from jax.experimental.pallas import tpu as pltpu
from jax.experimental.pallas import tpu_sc as plsc


def _make_pool(B, L, E, NC, NS, LANES):
    NW = NC * NS
    assert B % NW == 0 and E % LANES == 0
    b_per_w = B // NW
    NBUF = 4
    assert b_per_w % NBUF == 0
    n_quads = b_per_w // NBUF
    n_acc = E // LANES
    chunks = []
    off = 0
    while off < L:
        n = min(128, L - off)
        chunks.append((off, n))
        off += n

    mesh = plsc.VectorSubcoreMesh(core_axis_name="c", subcore_axis_name="s")

    @functools.partial(
        pl.kernel,
        out_type=jax.ShapeDtypeStruct((B, E), jnp.float32),
        mesh=mesh,
        compiler_params=pltpu.CompilerParams(use_tc_tiling_on_sc=False),
        scratch_types=[
            pltpu.VMEM((b_per_w, L), jnp.int32),
            pltpu.VMEM((L, E), jnp.float32),
            pltpu.VMEM((L, E), jnp.float32),
            pltpu.VMEM((L, E), jnp.float32),
            pltpu.VMEM((L, E), jnp.float32),
            pltpu.VMEM((b_per_w, E), jnp.float32),
            pltpu.SemaphoreType.DMA,
            pltpu.SemaphoreType.DMA,
            pltpu.SemaphoreType.DMA,
            pltpu.SemaphoreType.DMA,
        ],
    )
    def pool(ids_hbm, emb_hbm, out_hbm, idx_v,
             buf0, buf1, buf2, buf3, out_v, s0, s1, s2, s3):
        wid = lax.axis_index("s") * NC + lax.axis_index("c")
        base = wid * b_per_w
        bufs = (buf0, buf1, buf2, buf3)
        sems = (s0, s1, s2, s3)

        pltpu.sync_copy(ids_hbm.at[pl.ds(base, b_per_w)], idx_v)

        def descs(b, k):
            return [
                pltpu.make_async_copy(
                    emb_hbm.at[idx_v.at[b, pl.ds(off, n)]],
                    bufs[k].at[pl.ds(off, n)],
                    sems[k],
                )
                for off, n in chunks
            ]

        def issue(b, k):
            for cp in descs(b, k):
                cp.start()

        def drain(b, k):
            for cp in descs(b, k):
                cp.wait()

        def accumulate(b, k):
            buf = bufs[k]

            def acc_body(j, accs):
                return tuple(
                    accs[c] + buf[j, pl.ds(c * LANES, LANES)]
                    for c in range(n_acc)
                )

            accs = tuple(
                jnp.zeros((LANES,), jnp.float32) for _ in range(n_acc)
            )
            accs = lax.fori_loop(0, L, acc_body, accs, unroll=25)
            scale = jnp.float32(1.0 / L)
            for c in range(n_acc):
                out_v[b, pl.ds(c * LANES, LANES)] = accs[c] * scale

        for k in range(NBUF - 1):
            issue(k, k)

        def body(g, carry):
            b0 = NBUF * g
            for k in range(NBUF):
                b = b0 + k
                drain(b, k)

                @pl.when(b + NBUF - 1 < b_per_w)
                def _():
                    issue(b + NBUF - 1, (k + NBUF - 1) % NBUF)

                accumulate(b, k)
            return carry

        lax.fori_loop(0, n_quads, body, 0)
        pltpu.sync_copy(out_v, out_hbm.at[pl.ds(base, b_per_w)])

    return pool


def _mlp(pooled, W1, b1, W2, b2):
    B, E = pooled.shape
    H = W1.shape[0]
    O = W2.shape[0]
    OP = 128
    W2p = jnp.zeros((OP, H), W2.dtype).at[:O].set(W2)
    b2p = jnp.zeros((1, OP), b2.dtype).at[0, :O].set(b2)
    b1r = b1.reshape(1, H)
    BLK = 1024

    def body(x_ref, w1_ref, b1_ref, w2_ref, b2_ref, o_ref):
        x = x_ref[...]
        h = lax.dot_general(
            x, w1_ref[...], (((1,), (1,)), ((), ())),
            preferred_element_type=jnp.float32,
        ) + b1_ref[...]
        h = jnp.maximum(h, 0.0)
        o_ref[...] = lax.dot_general(
            h, w2_ref[...], (((1,), (1,)), ((), ())),
            preferred_element_type=jnp.float32,
        ) + b2_ref[...]

    out = pl.pallas_call(
        body,
        out_shape=jax.ShapeDtypeStruct((B, OP), jnp.float32),
        grid=(B // BLK,),
        in_specs=[
            pl.BlockSpec((BLK, E), lambda i: (i, 0)),
            pl.BlockSpec((H, E), lambda i: (0, 0)),
            pl.BlockSpec((1, H), lambda i: (0, 0)),
            pl.BlockSpec((OP, H), lambda i: (0, 0)),
            pl.BlockSpec((1, OP), lambda i: (0, 0)),
        ],
        out_specs=pl.BlockSpec((BLK, OP), lambda i: (i, 0)),
    )(pooled, W1, b1r, W2p, b2p)
    return out[:, :O]


def kernel(input_ids, emb, W1, b1, W2, b2):
    B, L = input_ids.shape
    V, E = emb.shape
    info = plsc.get_sparse_core_info()
    NC, NS, LANES = info.num_cores, info.num_subcores, info.num_lanes
    embp = jnp.pad(emb, ((0, 0), (0, 128 - E)))
    emb2 = embp.reshape(2 * V, E)
    pool = _make_pool(B, L, E, NC, NS, LANES)
    pooled = pool(input_ids.astype(jnp.int32) * 2, emb2)
    return _mlp(pooled, W1, b1, W2, b2)
